# baseline (speedup 1.0000x reference)
import jax
import jax.numpy as jnp
from jax.experimental import pallas as pl
from jax.experimental.pallas import tpu as pltpu

R = 256
NBUF = 4


def _body(idx_ref, h_ref, cv_ref, o_ref, *sc):
    ins = sc[0:NBUF]
    outs = sc[NBUF:2 * NBUF]
    isems = sc[2 * NBUF:3 * NBUF]
    osems = sc[3 * NBUF:4 * NBUF]
    B, S, E = h_ref.shape
    cpb = S // R
    shift = cpb.bit_length() - 1
    mask = cpb - 1
    nch = B * cpb

    def start_in(j, i):
        b = jnp.right_shift(i, shift)
        r0 = jnp.bitwise_and(i, mask) * R
        pltpu.make_async_copy(h_ref.at[b, pl.ds(r0, R)], ins[j], isems[j]).start()

    def wait_in(j):
        pltpu.make_async_copy(h_ref.at[0, pl.ds(0, R)], ins[j], isems[j]).wait()

    def start_out(j, i):
        b = jnp.right_shift(i, shift)
        r0 = jnp.bitwise_and(i, mask) * R
        pltpu.make_async_copy(outs[j], o_ref.at[b, pl.ds(r0, R)], osems[j]).start()

    def wait_out(j):
        pltpu.make_async_copy(outs[j], o_ref.at[0, pl.ds(0, R)], osems[j]).wait()

    def compute(j, i):
        b = jnp.right_shift(i, shift)
        adj = cv_ref[pl.ds(idx_ref[b], 1)]
        outs[j][...] = ins[j][...] + adj[0]

    for j in range(NBUF):
        start_in(j, j)

    for j in range(NBUF):
        wait_in(j)
        compute(j, j)
        start_out(j, j)
        start_in(j, j + NBUF)

    @pl.loop(NBUF, nch - NBUF, step=NBUF)
    def _(i):
        for j in range(NBUF):
            ii = i + j
            wait_in(j)
            wait_out(j)
            compute(j, ii)
            start_out(j, ii)
            start_in(j, ii + NBUF)

    for j in range(NBUF):
        ii = nch - NBUF + j
        wait_in(j)
        wait_out(j)
        compute(j, ii)
        start_out(j, ii)

    for j in range(NBUF):
        wait_out(j)


def kernel(hidden_states, affective_state_indices, control_vectors):
    B, S, E = hidden_states.shape
    n = control_vectors.shape[0]
    idx = jnp.clip(affective_state_indices.astype(jnp.int32), 0, n - 1)
    cv3 = control_vectors.reshape(n, 1, E)

    return pl.pallas_call(
        _body,
        grid_spec=pltpu.PrefetchScalarGridSpec(
            num_scalar_prefetch=1,
            grid=(1,),
            in_specs=[
                pl.BlockSpec(memory_space=pltpu.MemorySpace.HBM),
                pl.BlockSpec((n, 1, E), lambda g, idx_ref: (0, 0, 0)),
            ],
            out_specs=pl.BlockSpec(memory_space=pltpu.MemorySpace.HBM),
            scratch_shapes=[
                *([pltpu.VMEM((R, E), jnp.float32)] * (2 * NBUF)),
                *([pltpu.SemaphoreType.DMA] * (2 * NBUF)),
            ],
        ),
        out_shape=jax.ShapeDtypeStruct((B, S, E), hidden_states.dtype),
    )(idx, hidden_states, cv3)


# speedup vs baseline: 1.0207x; 1.0207x over previous
import jax
import jax.numpy as jnp
from jax.experimental import pallas as pl
from jax.experimental.pallas import tpu as pltpu

R = 512
NBUF = 4


def _body(idx_ref, h_ref, cv_ref, o_ref, *sc):
    ins = sc[0:NBUF]
    outs = sc[NBUF:2 * NBUF]
    isems = sc[2 * NBUF:3 * NBUF]
    osems = sc[3 * NBUF:4 * NBUF]
    B, S, E = h_ref.shape
    cpb = S // R
    shift = cpb.bit_length() - 1
    mask = cpb - 1
    nch = B * cpb

    def start_in(j, i):
        b = jnp.right_shift(i, shift)
        r0 = jnp.bitwise_and(i, mask) * R
        pltpu.make_async_copy(h_ref.at[b, pl.ds(r0, R)], ins[j], isems[j]).start()

    def wait_in(j):
        pltpu.make_async_copy(h_ref.at[0, pl.ds(0, R)], ins[j], isems[j]).wait()

    def start_out(j, i):
        b = jnp.right_shift(i, shift)
        r0 = jnp.bitwise_and(i, mask) * R
        pltpu.make_async_copy(outs[j], o_ref.at[b, pl.ds(r0, R)], osems[j]).start()

    def wait_out(j):
        pltpu.make_async_copy(outs[j], o_ref.at[0, pl.ds(0, R)], osems[j]).wait()

    def compute(j, i):
        b = jnp.right_shift(i, shift)
        adj = cv_ref[pl.ds(idx_ref[b], 1)]
        outs[j][...] = ins[j][...] + adj[0]

    for j in range(NBUF):
        start_in(j, j)

    for j in range(NBUF):
        wait_in(j)
        compute(j, j)
        start_out(j, j)
        start_in(j, j + NBUF)

    @pl.loop(NBUF, nch - NBUF, step=NBUF)
    def _(i):
        for j in range(NBUF):
            ii = i + j
            wait_in(j)
            wait_out(j)
            compute(j, ii)
            start_out(j, ii)
            start_in(j, ii + NBUF)

    for j in range(NBUF):
        ii = nch - NBUF + j
        wait_in(j)
        wait_out(j)
        compute(j, ii)
        start_out(j, ii)

    for j in range(NBUF):
        wait_out(j)


def kernel(hidden_states, affective_state_indices, control_vectors):
    B, S, E = hidden_states.shape
    n = control_vectors.shape[0]
    idx = jnp.clip(affective_state_indices.astype(jnp.int32), 0, n - 1)
    cv3 = control_vectors.reshape(n, 1, E)

    return pl.pallas_call(
        _body,
        grid_spec=pltpu.PrefetchScalarGridSpec(
            num_scalar_prefetch=1,
            grid=(1,),
            in_specs=[
                pl.BlockSpec(memory_space=pltpu.MemorySpace.HBM),
                pl.BlockSpec((n, 1, E), lambda g, idx_ref: (0, 0, 0)),
            ],
            out_specs=pl.BlockSpec(memory_space=pltpu.MemorySpace.HBM),
            scratch_shapes=[
                *([pltpu.VMEM((R, E), jnp.float32)] * (2 * NBUF)),
                *([pltpu.SemaphoreType.DMA] * (2 * NBUF)),
            ],
        ),
        out_shape=jax.ShapeDtypeStruct((B, S, E), hidden_states.dtype),
    )(idx, hidden_states, cv3)


# speedup vs baseline: 1.0334x; 1.0125x over previous
import jax
import jax.numpy as jnp
from jax.experimental import pallas as pl
from jax.experimental.pallas import tpu as pltpu


def _body(idx_ref, h_ref, cv_ref, o_ref):
    o_ref[...] = h_ref[...] + cv_ref[0]


def kernel(hidden_states, affective_state_indices, control_vectors):
    B, S, E = hidden_states.shape
    n = control_vectors.shape[0]
    idx = affective_state_indices.astype(jnp.int32)
    cv3 = control_vectors.reshape(n, 1, E)

    return pl.pallas_call(
        _body,
        grid_spec=pltpu.PrefetchScalarGridSpec(
            num_scalar_prefetch=1,
            grid=(B,),
            in_specs=[
                pl.BlockSpec((1, S, E), lambda b, idx_ref: (b, 0, 0)),
                pl.BlockSpec(
                    (1, 1, E),
                    lambda b, idx_ref: (jnp.clip(idx_ref[b], 0, n - 1), 0, 0),
                ),
            ],
            out_specs=pl.BlockSpec((1, S, E), lambda b, idx_ref: (b, 0, 0)),
        ),
        compiler_params=pltpu.CompilerParams(
            dimension_semantics=("arbitrary",),
        ),
        out_shape=jax.ShapeDtypeStruct((B, S, E), hidden_states.dtype),
    )(idx, hidden_states, cv3)


# speedup vs baseline: 1.0335x; 1.0001x over previous
import jax
import jax.numpy as jnp
from jax.experimental import pallas as pl
from jax.experimental.pallas import tpu as pltpu


def _body(idx_ref, h_ref, cv_ref, o_ref):
    o_ref[...] = h_ref[...] + cv_ref[0]


def kernel(hidden_states, affective_state_indices, control_vectors):
    B, S, E = hidden_states.shape
    n = control_vectors.shape[0]
    idx = affective_state_indices.astype(jnp.int32)
    cv3 = control_vectors.reshape(n, 1, E)
    BS = 2048
    grid = (B, S // BS)

    def h_map(b, s, idx_ref):
        return (b, s, 0)

    def cv_map(b, s, idx_ref):
        return (jnp.clip(idx_ref[b], 0, n - 1), 0, 0)

    return pl.pallas_call(
        _body,
        grid_spec=pltpu.PrefetchScalarGridSpec(
            num_scalar_prefetch=1,
            grid=grid,
            in_specs=[
                pl.BlockSpec((1, BS, E), h_map),
                pl.BlockSpec((1, 1, E), cv_map),
            ],
            out_specs=pl.BlockSpec((1, BS, E), h_map),
        ),
        out_shape=jax.ShapeDtypeStruct((B, S, E), hidden_states.dtype),
    )(idx, hidden_states, cv3)
